# Initial kernel scaffold; baseline (speedup 1.0000x reference)
import jax
import jax.numpy as jnp
from jax.experimental import pallas as pl


def kernel(seq_idx, pos_embed):
    raise NotImplementedError("write your pallas kernel here")


# speedup vs baseline: 3.4262x; 3.4262x over previous
import functools

import jax
import jax.numpy as jnp
from jax import lax
from jax.experimental import pallas as pl
from jax.experimental.pallas import tpu as pltpu
from jax.experimental.pallas import tpu_sc as plsc

_NUM_CORES = 2
_NUM_SUBCORES = 16
_NUM_WORKERS = _NUM_CORES * _NUM_SUBCORES

_D = 128
_CHUNK = 128


def _make_gather(n_total: int):
  n_per_w = n_total // _NUM_WORKERS
  assert n_per_w % _CHUNK == 0
  n_chunks = n_per_w // _CHUNK

  mesh = plsc.VectorSubcoreMesh(core_axis_name="c", subcore_axis_name="s")

  @functools.partial(
      pl.kernel,
      mesh=mesh,
      out_type=jax.ShapeDtypeStruct((n_total, _D), jnp.float32),
      scratch_types=[
          pltpu.VMEM((_CHUNK,), jnp.int32),
          pltpu.VMEM((_CHUNK, _D), jnp.float32),
          pltpu.SemaphoreType.DMA,
      ],
  )
  def gather_kernel(table_hbm, idx_hbm, out_hbm, idx_v, rows_v, sem):
    wid = lax.axis_index("s") * _NUM_CORES + lax.axis_index("c")
    base = wid * n_per_w

    def body(ch, _):
      start = base + ch * _CHUNK
      pltpu.sync_copy(idx_hbm.at[pl.ds(start, _CHUNK)], idx_v)
      pltpu.async_copy(table_hbm.at[idx_v], rows_v, sem).wait()
      pltpu.sync_copy(rows_v, out_hbm.at[pl.ds(start, _CHUNK)])
      return 0

    lax.fori_loop(0, n_chunks, body, 0)

  return gather_kernel


def kernel(seq_idx, pos_embed):
  batch, seq = seq_idx.shape
  n_total = batch * seq
  idx_flat = seq_idx.reshape(n_total)
  out = _make_gather(n_total)(pos_embed, idx_flat)
  return out.reshape(batch, seq, _D)


# speedup vs baseline: 3.4952x; 1.0201x over previous
import functools

import jax
import jax.numpy as jnp
from jax import lax
from jax.experimental import pallas as pl
from jax.experimental.pallas import tpu as pltpu
from jax.experimental.pallas import tpu_sc as plsc

_NUM_CORES = 2
_NUM_SUBCORES = 16
_NUM_WORKERS = _NUM_CORES * _NUM_SUBCORES

_D = 128
_CHUNK = 128
_NBUF = 2


def _make_gather(n_total: int):
  n_per_w = n_total // _NUM_WORKERS
  assert n_per_w % _CHUNK == 0
  n_chunks = n_per_w // _CHUNK
  assert n_chunks % _NBUF == 0 and n_chunks // _NBUF >= 2

  mesh = plsc.VectorSubcoreMesh(core_axis_name="c", subcore_axis_name="s")

  @functools.partial(
      pl.kernel,
      mesh=mesh,
      out_type=jax.ShapeDtypeStruct((n_total, _D), jnp.float32),
      scratch_types=[
          pltpu.VMEM((_CHUNK,), jnp.int32),
          pltpu.VMEM((_CHUNK,), jnp.int32),
          pltpu.VMEM((_CHUNK, _D), jnp.float32),
          pltpu.VMEM((_CHUNK, _D), jnp.float32),
          pltpu.SemaphoreType.DMA,
          pltpu.SemaphoreType.DMA,
          pltpu.SemaphoreType.DMA,
          pltpu.SemaphoreType.DMA,
      ],
  )
  def gather_kernel(table_hbm, idx_hbm, out_hbm,
                    idx0, idx1, rows0, rows1, g0, g1, o0, o1):
    wid = lax.axis_index("s") * _NUM_CORES + lax.axis_index("c")
    base = wid * n_per_w
    bufs = ((idx0, rows0, g0, o0), (idx1, rows1, g1, o1))

    def start_gather(ch, buf):
      idx_v, rows_v, gsem, _ = buf
      pltpu.sync_copy(idx_hbm.at[pl.ds(base + ch * _CHUNK, _CHUNK)], idx_v)
      pltpu.async_copy(table_hbm.at[idx_v], rows_v, gsem)

    def wait_gather(buf):
      idx_v, rows_v, gsem, _ = buf
      pltpu.make_async_copy(table_hbm.at[idx_v], rows_v, gsem).wait()

    def start_store(ch, buf):
      _, rows_v, _, osem = buf
      pltpu.async_copy(rows_v, out_hbm.at[pl.ds(base + ch * _CHUNK, _CHUNK)],
                       osem)

    def wait_store(ch, buf):
      _, rows_v, _, osem = buf
      pltpu.make_async_copy(
          rows_v, out_hbm.at[pl.ds(base + ch * _CHUNK, _CHUNK)], osem).wait()

    start_gather(0, bufs[0])
    start_gather(1, bufs[1])

    def body(g, _):
      for b in range(_NBUF):
        ch = _NBUF * g + b
        buf = bufs[b]
        wait_gather(buf)
        start_store(ch, buf)
        wait_store(ch, buf)
        start_gather(ch + _NBUF, buf)
      return 0

    lax.fori_loop(0, n_chunks // _NBUF - 1, body, 0)

    for b in range(_NBUF):
      ch = n_chunks - _NBUF + b
      buf = bufs[b]
      wait_gather(buf)
      start_store(ch, buf)
      wait_store(ch, buf)

  return gather_kernel


def kernel(seq_idx, pos_embed):
  batch, seq = seq_idx.shape
  n_total = batch * seq
  idx_flat = seq_idx.reshape(n_total)
  out = _make_gather(n_total)(pos_embed, idx_flat)
  return out.reshape(batch, seq, _D)
